# initial kernel scaffold (unmeasured)
import jax
import jax.numpy as jnp
from jax import lax
from jax.experimental import pallas as pl
from jax.experimental.pallas import tpu as pltpu

N_DEV = 16


def kernel(x, w_mat, scale_x, scale_w):
    m_per, k = x.shape
    _, n = w_mat.shape
    n_per = n // N_DEV
    m_tot = m_per * N_DEV

    def body(x_ref, w_ref, sx_ref, sw_ref, out_ref, comm_ref, send_sems, recv_sems):
        my_pos = lax.axis_index("i")
        scale = sx_ref[0] * sw_ref[0]
        x_val = x_ref[...]

        rdmas = []
        for off in range(1, N_DEV):
            tgt = lax.rem(my_pos + off, N_DEV)
            chunk = lax.dot_general(
                x_val,
                w_ref[:, pl.ds(tgt * n_per, n_per)],
                (((1,), (0,)), ((), ())),
                preferred_element_type=jnp.float32,
            )
            comm_ref[off - 1] = chunk * scale
            rdma = pltpu.make_async_remote_copy(
                src_ref=comm_ref.at[off - 1],
                dst_ref=out_ref.at[pl.ds(my_pos * m_per, m_per), :],
                send_sem=send_sems.at[off - 1],
                recv_sem=recv_sems.at[my_pos],
                device_id=(tgt,),
                device_id_type=pl.DeviceIdType.MESH,
            )
            rdma.start()
            rdmas.append(rdma)

        local = lax.dot_general(
            x_val,
            w_ref[:, pl.ds(my_pos * n_per, n_per)],
            (((1,), (0,)), ((), ())),
            preferred_element_type=jnp.float32,
        )
        out_ref[pl.ds(my_pos * m_per, m_per), :] = local * scale

        for rdma in rdmas:
            rdma.wait_send()
        for off in range(1, N_DEV):
            src = lax.rem(my_pos + off, N_DEV)
            recv = pltpu.make_async_remote_copy(
                src_ref=comm_ref.at[0],
                dst_ref=out_ref.at[pl.ds(src * m_per, m_per), :],
                send_sem=send_sems.at[0],
                recv_sem=recv_sems.at[src],
                device_id=(my_pos,),
                device_id_type=pl.DeviceIdType.MESH,
            )
            recv.wait_recv()

    return pl.pallas_call(
        body,
        out_shape=jax.ShapeDtypeStruct((m_tot, n_per), jnp.float32),
        in_specs=[
            pl.BlockSpec(memory_space=pltpu.VMEM),
            pl.BlockSpec(memory_space=pltpu.VMEM),
            pl.BlockSpec(memory_space=pltpu.SMEM),
            pl.BlockSpec(memory_space=pltpu.SMEM),
        ],
        out_specs=pl.BlockSpec(memory_space=pltpu.VMEM),
        scratch_shapes=[
            pltpu.VMEM((N_DEV - 1, m_per, n_per), jnp.float32),
            pltpu.SemaphoreType.DMA((N_DEV - 1,)),
            pltpu.SemaphoreType.DMA((N_DEV,)),
        ],
        compiler_params=pltpu.CompilerParams(
            vmem_limit_bytes=100 * 1024 * 1024,
        ),
    )(x, w_mat, scale_x, scale_w)


# baseline (device time: 112267 ns/iter reference)
import jax
import jax.numpy as jnp
from jax import lax
from jax.experimental import pallas as pl
from jax.experimental.pallas import tpu as pltpu

N_DEV = 16


def kernel(x, w_mat, scale_x, scale_w):
    m_per, k = x.shape
    _, n = w_mat.shape
    n_per = n // N_DEV
    m_tot = m_per * N_DEV

    def body(x_ref, w_ref, sx_ref, sw_ref, out_ref,
             xb_ref, wbuf_ref, comm_ref, dma_sems, send_sems, recv_sems):
        my_pos = lax.axis_index("i")
        scale = sx_ref[0] * sw_ref[0]

        def w_dma(slot, tgt):
            return pltpu.make_async_copy(
                w_ref.at[:, pl.ds(tgt * n_per, n_per)],
                wbuf_ref.at[slot],
                dma_sems.at[slot],
            )

        w_dma(0, lax.rem(my_pos + 1, N_DEV)).start()
        xb_ref[...] = x_ref[...].astype(jnp.bfloat16)

        rdmas = []
        for off in range(1, N_DEV):
            tgt = lax.rem(my_pos + off, N_DEV)
            cur = (off - 1) % 2
            nxt = off % 2
            nxt_tgt = lax.rem(my_pos + off + 1, N_DEV)
            w_dma(cur, tgt).wait()
            w_dma(nxt, nxt_tgt).start()

            chunk = lax.dot_general(
                xb_ref[...],
                wbuf_ref[cur].astype(jnp.bfloat16),
                (((1,), (0,)), ((), ())),
                preferred_element_type=jnp.float32,
            )
            comm_ref[off - 1] = chunk * scale
            rdma = pltpu.make_async_remote_copy(
                src_ref=comm_ref.at[off - 1],
                dst_ref=out_ref.at[pl.ds(my_pos * m_per, m_per), :],
                send_sem=send_sems.at[off - 1],
                recv_sem=recv_sems.at[my_pos],
                device_id=(tgt,),
                device_id_type=pl.DeviceIdType.MESH,
            )
            rdma.start()
            rdmas.append(rdma)

        last = (N_DEV - 1) % 2
        w_dma(last, my_pos).wait()
        local = lax.dot_general(
            xb_ref[...],
            wbuf_ref[last].astype(jnp.bfloat16),
            (((1,), (0,)), ((), ())),
            preferred_element_type=jnp.float32,
        )
        out_ref[pl.ds(my_pos * m_per, m_per), :] = local * scale

        for rdma in rdmas:
            rdma.wait_send()
        for off in range(1, N_DEV):
            src = lax.rem(my_pos + off, N_DEV)
            recv = pltpu.make_async_remote_copy(
                src_ref=comm_ref.at[0],
                dst_ref=out_ref.at[pl.ds(src * m_per, m_per), :],
                send_sem=send_sems.at[0],
                recv_sem=recv_sems.at[src],
                device_id=(my_pos,),
                device_id_type=pl.DeviceIdType.MESH,
            )
            recv.wait_recv()

    return pl.pallas_call(
        body,
        out_shape=jax.ShapeDtypeStruct((m_tot, n_per), jnp.float32),
        in_specs=[
            pl.BlockSpec(memory_space=pltpu.VMEM),
            pl.BlockSpec(memory_space=pl.ANY),
            pl.BlockSpec(memory_space=pltpu.SMEM),
            pl.BlockSpec(memory_space=pltpu.SMEM),
        ],
        out_specs=pl.BlockSpec(memory_space=pltpu.VMEM),
        scratch_shapes=[
            pltpu.VMEM((m_per, k), jnp.bfloat16),
            pltpu.VMEM((2, k, n_per), jnp.float32),
            pltpu.VMEM((N_DEV - 1, m_per, n_per), jnp.float32),
            pltpu.SemaphoreType.DMA((2,)),
            pltpu.SemaphoreType.DMA((N_DEV - 1,)),
            pltpu.SemaphoreType.DMA((N_DEV,)),
        ],
        compiler_params=pltpu.CompilerParams(
            vmem_limit_bytes=100 * 1024 * 1024,
        ),
    )(x, w_mat, scale_x, scale_w)
